# baseline (speedup 1.0000x reference)
import functools
import jax
import jax.numpy as jnp
from jax import lax
from jax.experimental import pallas as pl
from jax.experimental.pallas import tpu as pltpu

D_MODEL = 768
N_HEADS = 12
D_K = D_MODEL // N_HEADS
GRID = 24
N_TOK = GRID * GRID + 1
TOP_K = 32
BATCH = 8

_PREC = lax.Precision.HIGHEST
_NEG_INF = float("-inf")


def _proj_body(x_ref, wq_ref, wk_ref, wv_ref, bq_ref, bk_ref, bv_ref,
               q_ref, k_ref, v_ref, keep_ref):
    xb = x_ref[0]
    fulls = []
    for w_ref, b_ref, o_ref in ((wq_ref, bq_ref, q_ref),
                                (wk_ref, bk_ref, k_ref),
                                (wv_ref, bv_ref, v_ref)):
        full = lax.dot_general(xb, w_ref[...], (((1,), (1,)), ((), ())),
                               preferred_element_type=jnp.float32,
                               precision=lax.Precision.DEFAULT)
        full = full + b_ref[...]
        fulls.append(full)
        for h in range(N_HEADS):
            o_ref[0, h] = full[:, h * D_K:(h + 1) * D_K]

    full_q, full_k = fulls[0], fulls[1]
    hrow = lax.broadcasted_iota(jnp.int32, (N_HEADS, D_MODEL), 0)
    dlane = lax.broadcasted_iota(jnp.int32, (N_HEADS, D_MODEL), 1)
    hd_mask = (dlane // D_K) == hrow
    q0 = full_q[0:1, :]
    mq = jnp.where(hd_mask, q0, 0.0)
    qk0 = lax.dot_general(mq, full_k, (((1,), (1,)), ((), ())),
                          preferred_element_type=jnp.float32,
                          precision=_PREC)
    mh = jnp.where(hd_mask, 1.0, 0.0)
    kn0 = lax.dot_general(mh, full_k * full_k, (((1,), (1,)), ((), ())),
                          preferred_element_type=jnp.float32,
                          precision=_PREC)
    qn0 = jnp.sum(mq * q0, axis=1, keepdims=True)
    dist0 = qn0 + kn0 - 2.0 * qk0

    def topk_body(_, carry):
        cur, keep = carry
        m = jnp.min(cur, axis=1, keepdims=True)
        sel = cur == m
        return (jnp.where(sel, jnp.float32(3.4e38), cur),
                jnp.where(sel, 1.0, keep))

    _, keep = lax.fori_loop(
        0, TOP_K, topk_body,
        (dist0, jnp.zeros((N_HEADS, N_TOK), dtype=jnp.float32)))
    keep_ref[0, :, 0, :] = keep


def _qkv_project(x, W_q, b_q, W_k, b_k, W_v, b_v):
    B, L, D = x.shape
    grid = (B,)
    wspec = pl.BlockSpec((D, D), lambda b: (0, 0))
    bspec = pl.BlockSpec((1, D), lambda b: (0, 0))
    hspec = pl.BlockSpec((1, N_HEADS, L, D_K), lambda b: (b, 0, 0, 0))
    return pl.pallas_call(
        _proj_body,
        grid=grid,
        in_specs=[
            pl.BlockSpec((1, L, D), lambda b: (b, 0, 0)),
            wspec, wspec, wspec, bspec, bspec, bspec,
        ],
        out_specs=[hspec, hspec, hspec,
                   pl.BlockSpec((1, N_HEADS, 1, L), lambda b: (b, 0, 0, 0))],
        out_shape=[jax.ShapeDtypeStruct((B, N_HEADS, L, D_K), jnp.float32)] * 3
        + [jax.ShapeDtypeStruct((B, N_HEADS, 1, L), jnp.float32)],
    )(x, W_q, W_k, W_v, b_q.reshape(1, D), b_k.reshape(1, D),
      b_v.reshape(1, D))


def _attn_body(sig_ref, q_ref, k_ref, v_ref, keep_ref, wo_ref, bo_ref,
               w_out_ref, am_ref, out_ref):
    h = pl.program_id(1)
    L = N_TOK
    q = q_ref[0, 0]
    k = k_ref[0, 0]
    v = v_ref[0, 0]
    coef = -0.5 * jnp.exp(-2.0 * sig_ref[0])

    qn = jnp.sum(q * q, axis=1, keepdims=True)
    ones_row = jnp.ones((1, D_K), dtype=jnp.float32)
    kn_row = lax.dot_general(ones_row, k * k, (((1,), (1,)), ((), ())),
                             preferred_element_type=jnp.float32,
                             precision=_PREC)
    qk = lax.dot_general(q, k, (((1,), (1,)), ((), ())),
                         preferred_element_type=jnp.float32,
                         precision=_PREC)
    dist = qn + kn_row - 2.0 * qk
    dist = jnp.maximum(dist, 0.0)
    scores = coef * dist

    icol = lax.broadcasted_iota(jnp.int32, (L, 1), 0)
    jrow = lax.broadcasted_iota(jnp.int32, (1, L), 1)
    ri = (icol - 1) // GRID
    ci = (icol - 1) % GRID
    rj = (jrow - 1) // GRID
    cj = (jrow - 1) % GRID
    cheb = jnp.maximum(jnp.abs(ri - rj), jnp.abs(ci - cj)) <= 2
    valid = (icol == 0) | (jrow == 0) | (((icol > 0) & (jrow > 0)) & cheb)
    s_masked = jnp.where(valid, scores, _NEG_INF)

    keep0 = keep_ref[0, 0]
    s_final = jnp.where((icol == 0) & (keep0 <= 0.0), _NEG_INF, s_masked)

    m = jnp.max(s_final, axis=1, keepdims=True)
    e = jnp.exp(s_final - m)
    z = jnp.sum(e, axis=1, keepdims=True)
    w = e / z

    w_out_ref[0, 0] = w
    am_ref[0, 0] = (w > 1e-6).astype(jnp.float32)

    cons = lax.dot_general(w, v, (((1,), (0,)), ((), ())),
                           preferred_element_type=jnp.float32,
                           precision=lax.Precision.DEFAULT)
    po = lax.dot_general(cons, wo_ref[...], (((1,), (0,)), ((), ())),
                         preferred_element_type=jnp.float32,
                         precision=lax.Precision.DEFAULT)

    @pl.when(h == 0)
    def _init():
        out_ref[0] = po + bo_ref[...]

    @pl.when(h > 0)
    def _acc():
        out_ref[0] = out_ref[0] + po


def _attention(log_sigma, Q, K, V, keep, W_o, b_o):
    B, H, L, Dk = Q.shape
    D = D_MODEL
    hspec = pl.BlockSpec((1, 1, L, Dk), lambda b, h: (b, h, 0, 0))
    lspec = pl.BlockSpec((1, 1, L, L), lambda b, h: (b, h, 0, 0))
    return pl.pallas_call(
        _attn_body,
        grid=(B, H),
        in_specs=[
            pl.BlockSpec(memory_space=pltpu.SMEM),
            hspec, hspec, hspec,
            pl.BlockSpec((1, 1, 1, L), lambda b, h: (b, h, 0, 0)),
            pl.BlockSpec((Dk, D), lambda b, h: (h, 0)),
            pl.BlockSpec((1, D), lambda b, h: (0, 0)),
        ],
        out_specs=[
            lspec, lspec,
            pl.BlockSpec((1, L, D), lambda b, h: (b, 0, 0)),
        ],
        out_shape=[
            jax.ShapeDtypeStruct((B, H, L, L), jnp.float32),
            jax.ShapeDtypeStruct((B, H, L, L), jnp.float32),
            jax.ShapeDtypeStruct((B, L, D), jnp.float32),
        ],
    )(log_sigma.reshape(1), Q, K, V, keep, W_o.T, b_o.reshape(1, D))


@jax.jit
def kernel(x, W_q, b_q, W_k, b_k, W_v, b_v, W_o, b_o, log_sigma):
    Q, K, V, keep = _qkv_project(x, W_q, b_q, W_k, b_k, W_v, b_v)
    weights, amask, out = _attention(log_sigma, Q, K, V, keep, W_o, b_o)
    return out, weights, amask


# speedup vs baseline: 1.0356x; 1.0356x over previous
import functools
import jax
import jax.numpy as jnp
from jax import lax
from jax.experimental import pallas as pl
from jax.experimental.pallas import tpu as pltpu

D_MODEL = 768
N_HEADS = 12
D_K = D_MODEL // N_HEADS
GRID = 24
N_TOK = GRID * GRID + 1
TOP_K = 32
BATCH = 8

_PREC = lax.Precision.HIGHEST
_NEG_INF = float("-inf")


def _proj_body(x_ref, wq_ref, wk_ref, wv_ref, bq_ref, bk_ref, bv_ref,
               q_ref, k_ref, v_ref, keep_ref):
    xb = x_ref[0]
    fulls = []
    for w_ref, b_ref, o_ref in ((wq_ref, bq_ref, q_ref),
                                (wk_ref, bk_ref, k_ref),
                                (wv_ref, bv_ref, v_ref)):
        fullt = lax.dot_general(w_ref[...], xb, (((1,), (1,)), ((), ())),
                                preferred_element_type=jnp.float32,
                                precision=lax.Precision.DEFAULT)
        fullt = fullt + b_ref[...]
        fulls.append(fullt)
        for h in range(N_HEADS):
            o_ref[0, h] = fullt[h * D_K:(h + 1) * D_K, :]

    full_qt, full_kt = fulls[0], fulls[1]
    drow = lax.broadcasted_iota(jnp.int32, (D_MODEL, N_HEADS), 0)
    hlane = lax.broadcasted_iota(jnp.int32, (D_MODEL, N_HEADS), 1)
    hd_mask = (drow // D_K) == hlane
    q0c = full_qt[:, 0:1]
    mq = jnp.where(hd_mask, q0c, 0.0)
    qk0 = lax.dot_general(mq, full_kt, (((0,), (0,)), ((), ())),
                          preferred_element_type=jnp.float32,
                          precision=_PREC)
    kn0 = lax.dot_general(jnp.where(hd_mask, 1.0, 0.0),
                          full_kt * full_kt, (((0,), (0,)), ((), ())),
                          preferred_element_type=jnp.float32,
                          precision=_PREC)
    qn0 = lax.dot_general(mq, q0c, (((0,), (0,)), ((), ())),
                          preferred_element_type=jnp.float32,
                          precision=_PREC)
    dist0 = qn0 + kn0 - 2.0 * qk0

    def topk_body(_, carry):
        cur, keep = carry
        m = jnp.min(cur, axis=1, keepdims=True)
        sel = cur == m
        return (jnp.where(sel, jnp.float32(3.4e38), cur),
                jnp.where(sel, 1.0, keep))

    _, keep = lax.fori_loop(
        0, TOP_K, topk_body,
        (dist0, jnp.zeros((N_HEADS, N_TOK), dtype=jnp.float32)))
    keep_ref[0, :, 0, :] = keep


def _qkv_project(x, W_q, b_q, W_k, b_k, W_v, b_v):
    B, L, D = x.shape
    grid = (B,)
    wspec = pl.BlockSpec((D, D), lambda b: (0, 0))
    bspec = pl.BlockSpec((D, 1), lambda b: (0, 0))
    hspec = pl.BlockSpec((1, N_HEADS, D_K, L), lambda b: (b, 0, 0, 0))
    return pl.pallas_call(
        _proj_body,
        grid=grid,
        in_specs=[
            pl.BlockSpec((1, L, D), lambda b: (b, 0, 0)),
            wspec, wspec, wspec, bspec, bspec, bspec,
        ],
        out_specs=[hspec, hspec, hspec,
                   pl.BlockSpec((1, N_HEADS, 1, L), lambda b: (b, 0, 0, 0))],
        out_shape=[jax.ShapeDtypeStruct((B, N_HEADS, D_K, L), jnp.float32)] * 3
        + [jax.ShapeDtypeStruct((B, N_HEADS, 1, L), jnp.float32)],
    )(x, W_q, W_k, W_v, b_q.reshape(D, 1), b_k.reshape(D, 1),
      b_v.reshape(D, 1))


def _attn_body(sig_ref, q_ref, k_ref, v_ref, keep_ref, valid_ref, wo_ref,
               bo_ref, w_out_ref, am_ref, out_ref):
    h = pl.program_id(1)
    L = N_TOK
    qt = q_ref[0, 0]
    kt = k_ref[0, 0]
    vt = v_ref[0, 0]
    coef = -0.5 * jnp.exp(-2.0 * sig_ref[0])

    ones_col = jnp.ones((D_K, 1), dtype=jnp.float32)
    qn = lax.dot_general(qt * qt, ones_col, (((0,), (0,)), ((), ())),
                         preferred_element_type=jnp.float32,
                         precision=_PREC)
    kn_row = lax.dot_general(ones_col, kt * kt, (((0,), (0,)), ((), ())),
                             preferred_element_type=jnp.float32,
                             precision=_PREC)
    qk = lax.dot_general(qt, kt, (((0,), (0,)), ((), ())),
                         preferred_element_type=jnp.float32,
                         precision=_PREC)
    dist = qn + kn_row - 2.0 * qk
    scores = jnp.minimum(coef * dist, 0.0)

    keep0 = keep_ref[0, 0]
    icol = lax.broadcasted_iota(jnp.int32, (L, 1), 0)
    dead = (valid_ref[...] == 0.0) | ((icol == 0) & (keep0 <= 0.0))
    s_final = jnp.where(dead, _NEG_INF, scores)

    m = jnp.max(s_final, axis=1, keepdims=True)
    e = jnp.exp(s_final - m)
    z = jnp.sum(e, axis=1, keepdims=True)
    w = e * (1.0 / z)

    w_out_ref[0, 0] = w
    am_ref[0, 0] = (w > 1e-6).astype(jnp.float32)

    cons = lax.dot_general(w, vt, (((1,), (1,)), ((), ())),
                           preferred_element_type=jnp.float32,
                           precision=lax.Precision.DEFAULT)
    po = lax.dot_general(cons, wo_ref[...], (((1,), (0,)), ((), ())),
                         preferred_element_type=jnp.float32,
                         precision=lax.Precision.DEFAULT)

    @pl.when(h == 0)
    def _init():
        out_ref[0] = po + bo_ref[...]

    @pl.when(h > 0)
    def _acc():
        out_ref[0] = out_ref[0] + po


def _locality_valid_f32():
    import numpy as np
    N = N_TOK
    mask = np.zeros((N, N), dtype=np.float32)
    mask[0, :] = 1.0
    mask[:, 0] = 1.0
    idx = np.arange(GRID * GRID)
    r = idx // GRID
    c = idx % GRID
    m = (np.maximum(np.abs(r[:, None] - r[None, :]),
                    np.abs(c[:, None] - c[None, :])) <= 2)
    mask[1:, 1:] = m.astype(np.float32)
    return mask


_VALID_NP = _locality_valid_f32()


def _attention(log_sigma, Q, K, V, keep, W_o, b_o):
    B, H, Dk, L = Q.shape
    D = D_MODEL
    hspec = pl.BlockSpec((1, 1, Dk, L), lambda b, h: (b, h, 0, 0))
    lspec = pl.BlockSpec((1, 1, L, L), lambda b, h: (b, h, 0, 0))
    return pl.pallas_call(
        _attn_body,
        grid=(B, H),
        in_specs=[
            pl.BlockSpec(memory_space=pltpu.SMEM),
            hspec, hspec, hspec,
            pl.BlockSpec((1, 1, 1, L), lambda b, h: (b, h, 0, 0)),
            pl.BlockSpec((L, L), lambda b, h: (0, 0)),
            pl.BlockSpec((Dk, D), lambda b, h: (h, 0)),
            pl.BlockSpec((1, D), lambda b, h: (0, 0)),
        ],
        out_specs=[
            lspec, lspec,
            pl.BlockSpec((1, L, D), lambda b, h: (b, 0, 0)),
        ],
        out_shape=[
            jax.ShapeDtypeStruct((B, H, L, L), jnp.float32),
            jax.ShapeDtypeStruct((B, H, L, L), jnp.float32),
            jax.ShapeDtypeStruct((B, L, D), jnp.float32),
        ],
    )(log_sigma.reshape(1), Q, K, V, keep, jnp.asarray(_VALID_NP),
      W_o.T, b_o.reshape(1, D))


@jax.jit
def kernel(x, W_q, b_q, W_k, b_k, W_v, b_v, W_o, b_o, log_sigma):
    Q, K, V, keep = _qkv_project(x, W_q, b_q, W_k, b_k, W_v, b_v)
    weights, amask, out = _attention(log_sigma, Q, K, V, keep, W_o, b_o)
    return out, weights, amask
